# baseline (speedup 1.0000x reference)
import functools

import jax
import jax.numpy as jnp
from jax import lax
from jax.experimental import pallas as pl
from jax.experimental.pallas import tpu as pltpu
from jax.experimental.pallas import tpu_sc as plsc

N = 10000
E = 160000
NC = 2
NS = 16
IB1 = 56
IB2 = 128
NSLOT1 = 4
NSLOT2 = 4

NB1 = 180
E1P = NS * NB1 * IB1
NB2 = -(-(E // (NC * NS)) // IB2)
E2P = NC * NS * NB2 * IB2

NACC = 10112
CH = NACC // NS

_mesh = plsc.VectorSubcoreMesh(core_axis_name="c", subcore_axis_name="s")


def _agg_pipeline(tab_hbm, srcv, dstv, buf, acc, gsem, nb, nslot):
    ng = nb // nslot
    for b in range(nslot):
        pltpu.async_copy(tab_hbm.at[srcv.at[b]], buf.at[b], gsem[b])

    def group(gi, carry):
        for b in range(nslot):
            j = gi * nslot + b
            pltpu.make_async_copy(tab_hbm.at[srcv.at[j]], buf.at[b],
                                  gsem[b]).wait()
            pltpu.sync_copy(buf.at[b], acc.at[dstv.at[j]], add=True)

            @pl.when(gi < ng - 1)
            def _():
                pltpu.async_copy(tab_hbm.at[srcv.at[j + nslot]], buf.at[b],
                                 gsem[b])
        return carry

    lax.fori_loop(0, ng, group, 0)


@functools.partial(
    pl.kernel,
    out_type=jax.ShapeDtypeStruct((NC * NACC, 8), jnp.float32),
    mesh=_mesh,
    scratch_types=[
        pltpu.VMEM((NB2, IB2), jnp.int32),
        pltpu.VMEM((IB2, 8), jnp.float32),
        pltpu.VMEM_SHARED((NACC, 8), jnp.float32),
    ],
    compiler_params=pltpu.CompilerParams(use_tc_tiling_on_sc=False),
)
def _deg_sc(dst_hbm, ones_hbm, zero_hbm, out_hbm, dstv, onesv, acc):
    c = lax.axis_index("c")
    s = lax.axis_index("s")
    pltpu.sync_copy(dst_hbm.at[c, s], dstv)
    pltpu.sync_copy(ones_hbm, onesv)
    pltpu.sync_copy(zero_hbm.at[pl.ds(s * CH, CH)], acc.at[pl.ds(s * CH, CH)])
    plsc.subcore_barrier()

    def body(j, carry):
        pltpu.sync_copy(onesv, acc.at[dstv.at[j]], add=True)
        return carry

    lax.fori_loop(0, NB2, body, 0)
    plsc.subcore_barrier()
    pltpu.sync_copy(acc.at[pl.ds(s * CH, CH)],
                    out_hbm.at[pl.ds(c * NACC + s * CH, CH)])


@functools.partial(
    pl.kernel,
    out_type=jax.ShapeDtypeStruct((NC * NACC, 128), jnp.float32),
    mesh=_mesh,
    scratch_types=[
        pltpu.VMEM((NB1, IB1), jnp.int32),
        pltpu.VMEM((NB1, IB1), jnp.int32),
        pltpu.VMEM((NSLOT1, IB1, 128), jnp.float32),
        pltpu.VMEM_SHARED((NACC, 128), jnp.float32),
        [pltpu.SemaphoreType.DMA] * NSLOT1,
    ],
    compiler_params=pltpu.CompilerParams(use_tc_tiling_on_sc=False),
)
def _agg1_sc(tab_hbm, src_hbm, dst_hbm, zero_hbm, out_hbm,
             srcv, dstv, buf, acc, gsem):
    c = lax.axis_index("c")
    s = lax.axis_index("s")
    pltpu.sync_copy(src_hbm.at[c, s], srcv)
    pltpu.sync_copy(dst_hbm.at[s], dstv)
    pltpu.sync_copy(zero_hbm.at[pl.ds(s * CH, CH)], acc.at[pl.ds(s * CH, CH)])
    plsc.subcore_barrier()
    _agg_pipeline(tab_hbm, srcv, dstv, buf, acc, gsem, NB1, NSLOT1)
    plsc.subcore_barrier()
    pltpu.sync_copy(acc.at[pl.ds(s * CH, CH)],
                    out_hbm.at[pl.ds(c * NACC + s * CH, CH)])


@functools.partial(
    pl.kernel,
    out_type=jax.ShapeDtypeStruct((NC * NACC, 64), jnp.float32),
    mesh=_mesh,
    scratch_types=[
        pltpu.VMEM((NB2, IB2), jnp.int32),
        pltpu.VMEM((NB2, IB2), jnp.int32),
        pltpu.VMEM((NSLOT2, IB2, 64), jnp.float32),
        pltpu.VMEM_SHARED((NACC, 64), jnp.float32),
        [pltpu.SemaphoreType.DMA] * NSLOT2,
    ],
    compiler_params=pltpu.CompilerParams(use_tc_tiling_on_sc=False),
)
def _agg2_sc(tab_hbm, src_hbm, dst_hbm, zero_hbm, out_hbm,
             srcv, dstv, buf, acc, gsem):
    c = lax.axis_index("c")
    s = lax.axis_index("s")
    pltpu.sync_copy(src_hbm.at[c, s], srcv)
    pltpu.sync_copy(dst_hbm.at[c, s], dstv)
    pltpu.sync_copy(zero_hbm.at[pl.ds(s * CH, CH)], acc.at[pl.ds(s * CH, CH)])
    plsc.subcore_barrier()
    _agg_pipeline(tab_hbm, srcv, dstv, buf, acc, gsem, NB2, NSLOT2)
    plsc.subcore_barrier()
    pltpu.sync_copy(acc.at[pl.ds(s * CH, CH)],
                    out_hbm.at[pl.ds(c * NACC + s * CH, CH)])


_RB = NACC // NS


def _tc_b_body(x_ref, w1_ref, ph_ref, out_ref):
    ph = ph_ref[...]
    deg = 1.0 + ph[0] + ph[1]
    dinv = lax.rsqrt(deg[:, :1])
    h1 = jnp.dot(x_ref[...], w1_ref[...], preferred_element_type=jnp.float32)
    g1 = h1 * dinv
    out_ref[0] = g1[:, :128]
    out_ref[1] = g1[:, 128:]


def _tc_d_body(s1a_ref, s1b_ref, g1a_ref, g1b_ref, ph_ref, b1_ref, w2_ref,
               out_ref):
    ph = ph_ref[...]
    dinv = lax.rsqrt(1.0 + ph[0, :, :1] + ph[1, :, :1])
    lo = s1a_ref[...] + g1a_ref[0]
    hi = s1b_ref[...] + g1b_ref[0]
    pre = jnp.concatenate([lo, hi], axis=1) * dinv + b1_ref[...]
    a1 = jnp.maximum(pre, 0.0)
    h2 = jnp.dot(a1, w2_ref[...], preferred_element_type=jnp.float32)
    out_ref[...] = h2 * dinv


def _tc_f_body(s2a_ref, s2b_ref, g2_ref, ph_ref, b2_ref, out_ref):
    ph = ph_ref[...]
    dinv = lax.rsqrt(1.0 + ph[0, :, :1] + ph[1, :, :1])
    out_ref[...] = (s2a_ref[...] + s2b_ref[...] + g2_ref[...]) * dinv + b2_ref[...]


def kernel(x, edge_index, W1, b1, W2, b2):
    f32 = jnp.float32
    i32 = jnp.int32
    src = edge_index[0]
    dst = edge_index[1]

    pad1 = E1P - E
    trash1 = N + jnp.arange(pad1, dtype=i32) % (NACC - N)
    dst1 = jnp.concatenate([dst, trash1]).reshape(NS, NB1, IB1)
    src1 = jnp.concatenate([src, jnp.zeros((pad1,), i32)]).reshape(NS, NB1, IB1)
    src1 = src1[None] + (jnp.arange(NC, dtype=i32) * NACC)[:, None, None, None]

    pad2 = E2P - E
    trash2 = N + jnp.arange(pad2, dtype=i32) % (NACC - N)
    dst2 = jnp.concatenate([dst, trash2]).reshape(NC, NS, NB2, IB2)
    src2 = jnp.concatenate([src, jnp.zeros((pad2,), i32)]).reshape(NC, NS, NB2, IB2)

    ones8 = jnp.ones((IB2, 8), f32)
    z8 = jnp.zeros((NACC, 8), f32)
    z128 = jnp.zeros((NACC, 128), f32)
    z64 = jnp.zeros((NACC, 64), f32)

    phist = _deg_sc(dst2, ones8, z8).reshape(NC, NACC, 8)

    g1tab = pl.pallas_call(
        _tc_b_body,
        grid=(NS,),
        in_specs=[
            pl.BlockSpec((_RB, 256), lambda i: (i, 0)),
            pl.BlockSpec((256, 256), lambda i: (0, 0)),
            pl.BlockSpec((NC, _RB, 8), lambda i: (0, i, 0)),
        ],
        out_specs=pl.BlockSpec((NC, _RB, 128), lambda i: (0, i, 0)),
        out_shape=jax.ShapeDtypeStruct((NC, NACC, 128), f32),
    )(x, W1, phist)

    s1 = _agg1_sc(g1tab.reshape(NC * NACC, 128), src1, dst1, z128)

    g2 = pl.pallas_call(
        _tc_d_body,
        grid=(NS,),
        in_specs=[
            pl.BlockSpec((_RB, 128), lambda i: (i, 0)),
            pl.BlockSpec((_RB, 128), lambda i: (i + NS, 0)),
            pl.BlockSpec((1, _RB, 128), lambda i: (0, i, 0)),
            pl.BlockSpec((1, _RB, 128), lambda i: (1, i, 0)),
            pl.BlockSpec((NC, _RB, 8), lambda i: (0, i, 0)),
            pl.BlockSpec((1, 256), lambda i: (0, 0)),
            pl.BlockSpec((256, 64), lambda i: (0, 0)),
        ],
        out_specs=pl.BlockSpec((_RB, 64), lambda i: (i, 0)),
        out_shape=jax.ShapeDtypeStruct((NACC, 64), f32),
    )(s1, s1, g1tab, g1tab, phist, b1.reshape(1, 256), W2)

    s2 = _agg2_sc(g2, src2, dst2, z64)

    out = pl.pallas_call(
        _tc_f_body,
        grid=(NS,),
        in_specs=[
            pl.BlockSpec((_RB, 64), lambda i: (i, 0)),
            pl.BlockSpec((_RB, 64), lambda i: (i + NS, 0)),
            pl.BlockSpec((_RB, 64), lambda i: (i, 0)),
            pl.BlockSpec((NC, _RB, 8), lambda i: (0, i, 0)),
            pl.BlockSpec((1, 64), lambda i: (0, 0)),
        ],
        out_specs=pl.BlockSpec((_RB, 64), lambda i: (i, 0)),
        out_shape=jax.ShapeDtypeStruct((NACC, 64), f32),
    )(s2, s2, g2, phist, b2.reshape(1, 64))
    return out[:N]


# speedup vs baseline: 1.0804x; 1.0804x over previous
import functools

import jax
import jax.numpy as jnp
from jax import lax
from jax.experimental import pallas as pl
from jax.experimental.pallas import tpu as pltpu
from jax.experimental.pallas import tpu_sc as plsc

N = 10000
E = 160000
NC = 2
NS = 16
IB1 = 112
IB2 = 128
NSLOT1 = 6
NSLOT2 = 4

NB1 = 90
E1P = NS * NB1 * IB1
NB2 = -(-(E // (NC * NS)) // IB2)
E2P = NC * NS * NB2 * IB2

NACC = 10112
CH = NACC // NS

_mesh = plsc.VectorSubcoreMesh(core_axis_name="c", subcore_axis_name="s")


def _agg_pipeline(tab_hbm, srcv, dstv, buf, acc, gsem, nb, nslot):
    ng = nb // nslot
    for b in range(nslot):
        pltpu.async_copy(tab_hbm.at[srcv.at[b]], buf.at[b], gsem[b])

    def group(gi, carry):
        for b in range(nslot):
            j = gi * nslot + b
            pltpu.make_async_copy(tab_hbm.at[srcv.at[j]], buf.at[b],
                                  gsem[b]).wait()
            pltpu.sync_copy(buf.at[b], acc.at[dstv.at[j]], add=True)

            @pl.when(gi < ng - 1)
            def _():
                pltpu.async_copy(tab_hbm.at[srcv.at[j + nslot]], buf.at[b],
                                 gsem[b])
        return carry

    lax.fori_loop(0, ng, group, 0)


@functools.partial(
    pl.kernel,
    out_type=jax.ShapeDtypeStruct((NC * NACC, 8), jnp.float32),
    mesh=_mesh,
    scratch_types=[
        pltpu.VMEM((NB2, IB2), jnp.int32),
        pltpu.VMEM((IB2, 8), jnp.float32),
        pltpu.VMEM_SHARED((NACC, 8), jnp.float32),
    ],
    compiler_params=pltpu.CompilerParams(use_tc_tiling_on_sc=False),
)
def _deg_sc(dst_hbm, ones_hbm, zero_hbm, out_hbm, dstv, onesv, acc):
    c = lax.axis_index("c")
    s = lax.axis_index("s")
    pltpu.sync_copy(dst_hbm.at[c, s], dstv)
    pltpu.sync_copy(ones_hbm, onesv)
    pltpu.sync_copy(zero_hbm.at[pl.ds(s * CH, CH)], acc.at[pl.ds(s * CH, CH)])
    plsc.subcore_barrier()

    def body(j, carry):
        pltpu.sync_copy(onesv, acc.at[dstv.at[j]], add=True)
        return carry

    lax.fori_loop(0, NB2, body, 0)
    plsc.subcore_barrier()
    pltpu.sync_copy(acc.at[pl.ds(s * CH, CH)],
                    out_hbm.at[pl.ds(c * NACC + s * CH, CH)])


@functools.partial(
    pl.kernel,
    out_type=jax.ShapeDtypeStruct((NC * NACC, 128), jnp.bfloat16),
    mesh=_mesh,
    scratch_types=[
        pltpu.VMEM((NB1, IB1), jnp.int32),
        pltpu.VMEM((NB1, IB1), jnp.int32),
        pltpu.VMEM((NSLOT1, IB1, 128), jnp.bfloat16),
        pltpu.VMEM_SHARED((NACC, 128), jnp.bfloat16),
        [pltpu.SemaphoreType.DMA] * NSLOT1,
    ],
    compiler_params=pltpu.CompilerParams(use_tc_tiling_on_sc=False),
)
def _agg1_sc(tab_hbm, src_hbm, dst_hbm, zero_hbm, out_hbm,
             srcv, dstv, buf, acc, gsem):
    c = lax.axis_index("c")
    s = lax.axis_index("s")
    pltpu.sync_copy(src_hbm.at[c, s], srcv)
    pltpu.sync_copy(dst_hbm.at[s], dstv)
    pltpu.sync_copy(zero_hbm.at[pl.ds(s * CH, CH)], acc.at[pl.ds(s * CH, CH)])
    plsc.subcore_barrier()
    _agg_pipeline(tab_hbm, srcv, dstv, buf, acc, gsem, NB1, NSLOT1)
    plsc.subcore_barrier()
    pltpu.sync_copy(acc.at[pl.ds(s * CH, CH)],
                    out_hbm.at[pl.ds(c * NACC + s * CH, CH)])


@functools.partial(
    pl.kernel,
    out_type=jax.ShapeDtypeStruct((NC * NACC, 64), jnp.float32),
    mesh=_mesh,
    scratch_types=[
        pltpu.VMEM((NB2, IB2), jnp.int32),
        pltpu.VMEM((NB2, IB2), jnp.int32),
        pltpu.VMEM((NSLOT2, IB2, 64), jnp.float32),
        pltpu.VMEM_SHARED((NACC, 64), jnp.float32),
        [pltpu.SemaphoreType.DMA] * NSLOT2,
    ],
    compiler_params=pltpu.CompilerParams(use_tc_tiling_on_sc=False),
)
def _agg2_sc(tab_hbm, src_hbm, dst_hbm, zero_hbm, out_hbm,
             srcv, dstv, buf, acc, gsem):
    c = lax.axis_index("c")
    s = lax.axis_index("s")
    pltpu.sync_copy(src_hbm.at[c, s], srcv)
    pltpu.sync_copy(dst_hbm.at[c, s], dstv)
    pltpu.sync_copy(zero_hbm.at[pl.ds(s * CH, CH)], acc.at[pl.ds(s * CH, CH)])
    plsc.subcore_barrier()
    _agg_pipeline(tab_hbm, srcv, dstv, buf, acc, gsem, NB2, NSLOT2)
    plsc.subcore_barrier()
    pltpu.sync_copy(acc.at[pl.ds(s * CH, CH)],
                    out_hbm.at[pl.ds(c * NACC + s * CH, CH)])


_RB = NACC // NS


def _tc_b_body(x_ref, w1_ref, ph_ref, out_ref):
    ph = ph_ref[...]
    deg = 1.0 + ph[0] + ph[1]
    dinv = lax.rsqrt(deg[:, :1])
    h1 = jnp.dot(x_ref[...], w1_ref[...], preferred_element_type=jnp.float32)
    g1 = (h1 * dinv).astype(jnp.bfloat16)
    out_ref[0] = g1[:, :128]
    out_ref[1] = g1[:, 128:]


def _tc_d_body(s1a_ref, s1b_ref, g1a_ref, g1b_ref, ph_ref, b1_ref, w2_ref,
               out_ref):
    ph = ph_ref[...]
    dinv = lax.rsqrt(1.0 + ph[0, :, :1] + ph[1, :, :1])
    lo = s1a_ref[...].astype(jnp.float32) + g1a_ref[0].astype(jnp.float32)
    hi = s1b_ref[...].astype(jnp.float32) + g1b_ref[0].astype(jnp.float32)
    pre = jnp.concatenate([lo, hi], axis=1) * dinv + b1_ref[...]
    a1 = jnp.maximum(pre, 0.0)
    h2 = jnp.dot(a1, w2_ref[...], preferred_element_type=jnp.float32)
    out_ref[...] = h2 * dinv


def _tc_f_body(s2a_ref, s2b_ref, g2_ref, ph_ref, b2_ref, out_ref):
    ph = ph_ref[...]
    dinv = lax.rsqrt(1.0 + ph[0, :, :1] + ph[1, :, :1])
    out_ref[...] = (s2a_ref[...] + s2b_ref[...] + g2_ref[...]) * dinv + b2_ref[...]


def kernel(x, edge_index, W1, b1, W2, b2):
    f32 = jnp.float32
    i32 = jnp.int32
    src = edge_index[0]
    dst = edge_index[1]

    pad1 = E1P - E
    trash1 = N + jnp.arange(pad1, dtype=i32) % (NACC - N)
    dst1 = jnp.concatenate([dst, trash1]).reshape(NS, NB1, IB1)
    src1 = jnp.concatenate([src, jnp.zeros((pad1,), i32)]).reshape(NS, NB1, IB1)
    src1 = src1[None] + (jnp.arange(NC, dtype=i32) * NACC)[:, None, None, None]

    pad2 = E2P - E
    trash2 = N + jnp.arange(pad2, dtype=i32) % (NACC - N)
    dst2 = jnp.concatenate([dst, trash2]).reshape(NC, NS, NB2, IB2)
    src2 = jnp.concatenate([src, jnp.zeros((pad2,), i32)]).reshape(NC, NS, NB2, IB2)

    ones8 = jnp.ones((IB2, 8), f32)
    z8 = jnp.zeros((NACC, 8), f32)
    z128 = jnp.zeros((NACC, 128), jnp.bfloat16)
    z64 = jnp.zeros((NACC, 64), f32)

    phist = _deg_sc(dst2, ones8, z8).reshape(NC, NACC, 8)

    g1tab = pl.pallas_call(
        _tc_b_body,
        grid=(NS,),
        in_specs=[
            pl.BlockSpec((_RB, 256), lambda i: (i, 0)),
            pl.BlockSpec((256, 256), lambda i: (0, 0)),
            pl.BlockSpec((NC, _RB, 8), lambda i: (0, i, 0)),
        ],
        out_specs=pl.BlockSpec((NC, _RB, 128), lambda i: (0, i, 0)),
        out_shape=jax.ShapeDtypeStruct((NC, NACC, 128), jnp.bfloat16),
    )(x, W1, phist)

    s1 = _agg1_sc(g1tab.reshape(NC * NACC, 128), src1, dst1, z128)

    g2 = pl.pallas_call(
        _tc_d_body,
        grid=(NS,),
        in_specs=[
            pl.BlockSpec((_RB, 128), lambda i: (i, 0)),
            pl.BlockSpec((_RB, 128), lambda i: (i + NS, 0)),
            pl.BlockSpec((1, _RB, 128), lambda i: (0, i, 0)),
            pl.BlockSpec((1, _RB, 128), lambda i: (1, i, 0)),
            pl.BlockSpec((NC, _RB, 8), lambda i: (0, i, 0)),
            pl.BlockSpec((1, 256), lambda i: (0, 0)),
            pl.BlockSpec((256, 64), lambda i: (0, 0)),
        ],
        out_specs=pl.BlockSpec((_RB, 64), lambda i: (i, 0)),
        out_shape=jax.ShapeDtypeStruct((NACC, 64), f32),
    )(s1, s1, g1tab, g1tab, phist, b1.reshape(1, 256), W2)

    s2 = _agg2_sc(g2, src2, dst2, z64)

    out = pl.pallas_call(
        _tc_f_body,
        grid=(NS,),
        in_specs=[
            pl.BlockSpec((_RB, 64), lambda i: (i, 0)),
            pl.BlockSpec((_RB, 64), lambda i: (i + NS, 0)),
            pl.BlockSpec((_RB, 64), lambda i: (i, 0)),
            pl.BlockSpec((NC, _RB, 8), lambda i: (0, i, 0)),
            pl.BlockSpec((1, 64), lambda i: (0, 0)),
        ],
        out_specs=pl.BlockSpec((_RB, 64), lambda i: (i, 0)),
        out_shape=jax.ShapeDtypeStruct((NACC, 64), f32),
    )(s2, s2, g2, phist, b2.reshape(1, 64))
    return out[:N]


# speedup vs baseline: 1.3081x; 1.2108x over previous
import functools

import jax
import jax.numpy as jnp
from jax import lax
from jax.experimental import pallas as pl
from jax.experimental.pallas import tpu as pltpu
from jax.experimental.pallas import tpu_sc as plsc

N = 10000
E = 160000
NC = 2
NS = 16
IB1 = 112
IB2 = 128
NSLOT1 = 6
NSLOT2 = 4

NB1 = 90
E1P = NS * NB1 * IB1
NB2 = -(-(E // (NC * NS)) // IB2)
E2P = NC * NS * NB2 * IB2

NACC = 10112
CH = NACC // NS

_mesh = plsc.VectorSubcoreMesh(core_axis_name="c", subcore_axis_name="s")


def _agg_pipeline(tab_hbm, srcv, dstv, buf, acc, gsem, nb, nslot):
    ng = nb // nslot
    for b in range(nslot):
        pltpu.async_copy(tab_hbm.at[srcv.at[b]], buf.at[b], gsem[b])

    def group(gi, carry):
        for b in range(nslot):
            j = gi * nslot + b
            pltpu.make_async_copy(tab_hbm.at[srcv.at[j]], buf.at[b],
                                  gsem[b]).wait()
            pltpu.sync_copy(buf.at[b], acc.at[dstv.at[j]], add=True)

            @pl.when(gi < ng - 1)
            def _():
                pltpu.async_copy(tab_hbm.at[srcv.at[j + nslot]], buf.at[b],
                                 gsem[b])
        return carry

    lax.fori_loop(0, ng, group, 0)


@functools.partial(
    pl.kernel,
    out_type=jax.ShapeDtypeStruct((NC * NACC, 8), jnp.float32),
    mesh=_mesh,
    scratch_types=[
        pltpu.VMEM((NB2, IB2), jnp.int32),
        pltpu.VMEM((IB2, 8), jnp.float32),
        pltpu.VMEM_SHARED((NACC, 8), jnp.float32),
    ],
    compiler_params=pltpu.CompilerParams(use_tc_tiling_on_sc=False),
)
def _deg_sc(dst_hbm, ones_hbm, zero_hbm, out_hbm, dstv, onesv, acc):
    c = lax.axis_index("c")
    s = lax.axis_index("s")
    pltpu.sync_copy(dst_hbm.at[c, s], dstv)
    pltpu.sync_copy(ones_hbm, onesv)
    pltpu.sync_copy(zero_hbm.at[pl.ds(s * CH, CH)], acc.at[pl.ds(s * CH, CH)])
    plsc.subcore_barrier()

    def body(j, carry):
        pltpu.sync_copy(onesv, acc.at[dstv.at[j]], add=True)
        return carry

    lax.fori_loop(0, NB2, body, 0)
    plsc.subcore_barrier()
    pltpu.sync_copy(acc.at[pl.ds(s * CH, CH)],
                    out_hbm.at[pl.ds(c * NACC + s * CH, CH)])


@functools.partial(
    pl.kernel,
    out_type=jax.ShapeDtypeStruct((NC * NACC, 128), jnp.bfloat16),
    mesh=_mesh,
    scratch_types=[
        pltpu.VMEM((NB1, IB1), jnp.int32),
        pltpu.VMEM((NB1, IB1), jnp.int32),
        pltpu.VMEM((NSLOT1, IB1, 128), jnp.bfloat16),
        pltpu.VMEM_SHARED((NACC, 128), jnp.bfloat16),
        [pltpu.SemaphoreType.DMA] * NSLOT1,
    ],
    compiler_params=pltpu.CompilerParams(use_tc_tiling_on_sc=False),
)
def _agg1_sc(tab_hbm, src_hbm, dst_hbm, zero_hbm, out_hbm,
             srcv, dstv, buf, acc, gsem):
    c = lax.axis_index("c")
    s = lax.axis_index("s")
    pltpu.sync_copy(src_hbm.at[c, s], srcv)
    pltpu.sync_copy(dst_hbm.at[s], dstv)
    pltpu.sync_copy(zero_hbm.at[pl.ds(s * CH, CH)], acc.at[pl.ds(s * CH, CH)])
    plsc.subcore_barrier()
    _agg_pipeline(tab_hbm, srcv, dstv, buf, acc, gsem, NB1, NSLOT1)
    plsc.subcore_barrier()
    pltpu.sync_copy(acc.at[pl.ds(s * CH, CH)],
                    out_hbm.at[pl.ds(c * NACC + s * CH, CH)])


@functools.partial(
    pl.kernel,
    out_type=jax.ShapeDtypeStruct((NC * NACC, 64), jnp.bfloat16),
    mesh=_mesh,
    scratch_types=[
        pltpu.VMEM((NB2, IB2), jnp.int32),
        pltpu.VMEM((NB2, IB2), jnp.int32),
        pltpu.VMEM((NSLOT2, IB2, 64), jnp.bfloat16),
        pltpu.VMEM_SHARED((NACC, 64), jnp.bfloat16),
        [pltpu.SemaphoreType.DMA] * NSLOT2,
    ],
    compiler_params=pltpu.CompilerParams(use_tc_tiling_on_sc=False),
)
def _agg2_sc(tab_hbm, src_hbm, dst_hbm, zero_hbm, out_hbm,
             srcv, dstv, buf, acc, gsem):
    c = lax.axis_index("c")
    s = lax.axis_index("s")
    pltpu.sync_copy(src_hbm.at[c, s], srcv)
    pltpu.sync_copy(dst_hbm.at[c, s], dstv)
    pltpu.sync_copy(zero_hbm.at[pl.ds(s * CH, CH)], acc.at[pl.ds(s * CH, CH)])
    plsc.subcore_barrier()
    _agg_pipeline(tab_hbm, srcv, dstv, buf, acc, gsem, NB2, NSLOT2)
    plsc.subcore_barrier()
    pltpu.sync_copy(acc.at[pl.ds(s * CH, CH)],
                    out_hbm.at[pl.ds(c * NACC + s * CH, CH)])


_RB = NACC // NS


def _tc_b_body(x_ref, w1_ref, ph_ref, out_ref):
    ph = ph_ref[...]
    deg = 1.0 + ph[0] + ph[1]
    dinv = lax.rsqrt(deg[:, :1])
    h1 = jnp.dot(x_ref[...], w1_ref[...], preferred_element_type=jnp.float32)
    g1 = (h1 * dinv).astype(jnp.bfloat16)
    out_ref[0] = g1[:, :128]
    out_ref[1] = g1[:, 128:]


def _tc_d_body(s1a_ref, s1b_ref, g1a_ref, g1b_ref, ph_ref, b1_ref, w2_ref,
               out_ref):
    ph = ph_ref[...]
    dinv = lax.rsqrt(1.0 + ph[0, :, :1] + ph[1, :, :1])
    lo = s1a_ref[...].astype(jnp.float32) + g1a_ref[0].astype(jnp.float32)
    hi = s1b_ref[...].astype(jnp.float32) + g1b_ref[0].astype(jnp.float32)
    pre = jnp.concatenate([lo, hi], axis=1) * dinv + b1_ref[...]
    a1 = jnp.maximum(pre, 0.0)
    h2 = jnp.dot(a1, w2_ref[...], preferred_element_type=jnp.float32)
    out_ref[...] = (h2 * dinv).astype(jnp.bfloat16)


def _tc_f_body(s2a_ref, s2b_ref, g2_ref, ph_ref, b2_ref, out_ref):
    ph = ph_ref[...]
    dinv = lax.rsqrt(1.0 + ph[0, :, :1] + ph[1, :, :1])
    s2 = (s2a_ref[...].astype(jnp.float32) + s2b_ref[...].astype(jnp.float32)
          + g2_ref[...].astype(jnp.float32))
    out_ref[...] = s2 * dinv + b2_ref[...]


def kernel(x, edge_index, W1, b1, W2, b2):
    f32 = jnp.float32
    i32 = jnp.int32
    src = edge_index[0]
    dst = edge_index[1]

    pad1 = E1P - E
    trash1 = N + jnp.arange(pad1, dtype=i32) % (NACC - N)
    dst1 = jnp.concatenate([dst, trash1]).reshape(NS, NB1, IB1)
    src1 = jnp.concatenate([src, jnp.zeros((pad1,), i32)]).reshape(NS, NB1, IB1)
    src1 = src1[None] + (jnp.arange(NC, dtype=i32) * NACC)[:, None, None, None]

    pad2 = E2P - E
    trash2 = N + jnp.arange(pad2, dtype=i32) % (NACC - N)
    dst2 = jnp.concatenate([dst, trash2]).reshape(NC, NS, NB2, IB2)
    src2 = jnp.concatenate([src, jnp.zeros((pad2,), i32)]).reshape(NC, NS, NB2, IB2)

    ones8 = jnp.ones((IB2, 8), f32)
    z8 = jnp.zeros((NACC, 8), f32)
    z128 = jnp.zeros((NACC, 128), jnp.bfloat16)
    z64 = jnp.zeros((NACC, 64), jnp.bfloat16)

    phist = _deg_sc(dst2, ones8, z8).reshape(NC, NACC, 8)

    g1tab = pl.pallas_call(
        _tc_b_body,
        grid=(NS,),
        in_specs=[
            pl.BlockSpec((_RB, 256), lambda i: (i, 0)),
            pl.BlockSpec((256, 256), lambda i: (0, 0)),
            pl.BlockSpec((NC, _RB, 8), lambda i: (0, i, 0)),
        ],
        out_specs=pl.BlockSpec((NC, _RB, 128), lambda i: (0, i, 0)),
        out_shape=jax.ShapeDtypeStruct((NC, NACC, 128), jnp.bfloat16),
    )(x, W1, phist)

    s1 = _agg1_sc(g1tab.reshape(NC * NACC, 128), src1, dst1, z128)

    g2 = pl.pallas_call(
        _tc_d_body,
        grid=(NS,),
        in_specs=[
            pl.BlockSpec((_RB, 128), lambda i: (i, 0)),
            pl.BlockSpec((_RB, 128), lambda i: (i + NS, 0)),
            pl.BlockSpec((1, _RB, 128), lambda i: (0, i, 0)),
            pl.BlockSpec((1, _RB, 128), lambda i: (1, i, 0)),
            pl.BlockSpec((NC, _RB, 8), lambda i: (0, i, 0)),
            pl.BlockSpec((1, 256), lambda i: (0, 0)),
            pl.BlockSpec((256, 64), lambda i: (0, 0)),
        ],
        out_specs=pl.BlockSpec((_RB, 64), lambda i: (i, 0)),
        out_shape=jax.ShapeDtypeStruct((NACC, 64), jnp.bfloat16),
    )(s1, s1, g1tab, g1tab, phist, b1.reshape(1, 256), W2)

    s2 = _agg2_sc(g2, src2, dst2, z64)

    out = pl.pallas_call(
        _tc_f_body,
        grid=(NS,),
        in_specs=[
            pl.BlockSpec((_RB, 64), lambda i: (i, 0)),
            pl.BlockSpec((_RB, 64), lambda i: (i + NS, 0)),
            pl.BlockSpec((_RB, 64), lambda i: (i, 0)),
            pl.BlockSpec((NC, _RB, 8), lambda i: (0, i, 0)),
            pl.BlockSpec((1, 64), lambda i: (0, 0)),
        ],
        out_specs=pl.BlockSpec((_RB, 64), lambda i: (i, 0)),
        out_shape=jax.ShapeDtypeStruct((NACC, 64), f32),
    )(s2, s2, g2, phist, b2.reshape(1, 64))
    return out[:N]
